# baseline (speedup 1.0000x reference)
import functools

import numpy as np

import jax
import jax.numpy as jnp
from jax import lax
from jax.experimental import pallas as pl
from jax.experimental.pallas import tpu as pltpu


def _tile(dim, candidates=(64, 32, 16, 8)):
    for t in candidates:
        if dim % t == 0:
            return t
    return dim


def _resize_matrix(n_in, n_out):
    o = np.arange(n_out, dtype=np.float64)
    src = (o + 0.5) * (n_in / n_out) - 0.5
    lo = np.floor(src).astype(np.int64)
    frac = (src - lo).astype(np.float32)
    lo_c = np.clip(lo, 0, n_in - 1)
    hi_c = np.clip(lo + 1, 0, n_in - 1)
    mat = np.zeros((n_out, n_in), dtype=np.float32)
    mat[np.arange(n_out), lo_c] += 1.0 - frac
    mat[np.arange(n_out), hi_c] += frac
    return mat


def _feat_kernel(wf_ref, wm_ref, x_ref, y_ref, s_ref, o_ref):
    c_in = x_ref.shape[1]
    h = x_ref.shape[2]
    w = x_ref.shape[3]
    hs = h // 3
    c_fea = wf_ref.shape[1]
    sel = s_ref[...]

    def feat(img_ref, w_off):
        x = img_ref[0].reshape(c_in, hs, 3, w)[:, :, 0, :]
        x = x.reshape(c_in * hs, w)
        xs = jnp.dot(x, sel, preferred_element_type=jnp.float32)
        acc = None
        for co in range(c_fea):
            f = wf_ref[0, co] * xs[0:hs]
            for ci in range(1, c_in):
                f = f + wf_ref[ci, co] * xs[ci * hs:(ci + 1) * hs]
            t = wm_ref[w_off + co, 0] * jnp.maximum(f, 0.0)
            acc = t if acc is None else acc + t
        return acc

    o_ref[0, 0] = feat(x_ref, 0)
    o_ref[1, 0] = feat(y_ref, c_fea)


def _features(x_img, y_img, w_fea, w_mat):
    n, c_in, h, w = x_img.shape
    hs, ws = h // 3, w // 3
    sel = np.zeros((w, ws), dtype=np.float32)
    sel[3 * np.arange(ws), np.arange(ws)] = 1.0
    return pl.pallas_call(
        _feat_kernel,
        out_shape=jax.ShapeDtypeStruct((2, n, hs, ws), jnp.float32),
        grid=(n, hs // 64),
        in_specs=[
            pl.BlockSpec(memory_space=pltpu.MemorySpace.SMEM),
            pl.BlockSpec(memory_space=pltpu.MemorySpace.SMEM),
            pl.BlockSpec((1, c_in, 192, w), lambda b, hh: (b, 0, hh, 0)),
            pl.BlockSpec((1, c_in, 192, w), lambda b, hh: (b, 0, hh, 0)),
            pl.BlockSpec((w, ws), lambda b, hh: (0, 0)),
        ],
        out_specs=pl.BlockSpec((2, 1, 64, ws), lambda b, hh: (0, b, hh, 0)),
        compiler_params=pltpu.CompilerParams(
            dimension_semantics=("parallel", "parallel")),
    )(w_fea, w_mat, x_img, y_img, jnp.asarray(sel))


def _disp_kernel(px_ref, py_ref, uh_ref, uw_ref, o_ref, *, d_low):
    px = px_ref[0, 0]
    py = py_ref[0, 0]
    uh = uh_ref[...]
    uw = uw_ref[...]
    t3, hs = uh.shape
    ws, w3 = uw.shape

    a = jnp.dot(uh, px, preferred_element_type=jnp.float32)
    b = jnp.dot(uh, py, preferred_element_type=jnp.float32)

    m_s = jnp.min(a) + jnp.min(b)
    a = a - m_s

    wpos = lax.broadcasted_iota(jnp.int32, (t3, ws), 1)
    neg_ms = -m_s
    cd = 4 if d_low % 4 == 0 else 1
    pp = pc = None
    num = jnp.zeros((t3, w3), jnp.float32)
    den = jnp.zeros((t3, w3), jnp.float32)

    def level(d, ppv, pcv, pnv, num, den):
        p2 = pcv * pcv
        t = ppv + pcv + pnv
        num = num + p2 * ((3.0 * d + 1.0) * t + (pnv - ppv))
        den = den + p2 * t
        return num, den

    for c in range(0, d_low, cd):
        rows = []
        for j in range(cd):
            d = c + j
            r = b if d == 0 else jnp.roll(b, d, axis=1)
            rows.append(jnp.where(wpos >= d, a + r, neg_ms).astype(jnp.bfloat16))
        xc = jnp.concatenate(rows, axis=0)
        q4 = jnp.dot(xc, uw, preferred_element_type=jnp.float32)
        q4 = q4.reshape(cd, t3, w3)
        for j in range(cd):
            pn = jnp.exp2(q4[j])
            if pc is None:
                pp = pc = pn
                continue
            num, den = level(c + j - 1, pp, pc, pn, num, den)
            pp = pc
            pc = pn
    num, den = level(d_low - 1, pp, pc, pc, num, den)
    o_ref[0] = num / den


def _disp(fea2, maxdisp):
    _, n, hs, ws = fea2.shape
    d_low = maxdisp // 3
    h3, w3 = hs * 3, ws * 3
    t3 = _tile(h3, candidates=(384, 192, 128, 64, 32, 16, 8))
    uh = jnp.asarray(_resize_matrix(hs, h3))
    uw = jnp.asarray(_resize_matrix(ws, w3).T
                     * (-1.4426950408889634 / 3.0))
    return pl.pallas_call(
        functools.partial(_disp_kernel, d_low=d_low),
        out_shape=jax.ShapeDtypeStruct((n, h3, w3), jnp.float32),
        grid=(n, h3 // t3),
        in_specs=[
            pl.BlockSpec((1, 1, hs, ws), lambda bb, hh: (0, bb, 0, 0)),
            pl.BlockSpec((1, 1, hs, ws), lambda bb, hh: (1, bb, 0, 0)),
            pl.BlockSpec((t3, hs), lambda bb, hh: (hh, 0)),
            pl.BlockSpec((ws, w3), lambda bb, hh: (0, 0)),
        ],
        out_specs=pl.BlockSpec((1, t3, w3), lambda bb, hh: (bb, hh, 0)),
        compiler_params=pltpu.CompilerParams(
            dimension_semantics=("parallel", "parallel")),
    )(fea2, fea2, uh, jnp.asarray(uw, jnp.bfloat16))


@functools.partial(jax.jit, static_argnames=("maxdisp",))
def _forward(x_img, y_img, w_fea, w_mat, *, maxdisp):
    fea2 = _features(x_img, y_img, w_fea, w_mat)
    return _disp(fea2, maxdisp)


def kernel(x_img, y_img, w_fea, w_mat):
    return _forward(x_img, y_img, w_fea, w_mat, maxdisp=192)


# speedup vs baseline: 1.1388x; 1.1388x over previous
import functools

import numpy as np

import jax
import jax.numpy as jnp
from jax import lax
from jax.experimental import pallas as pl
from jax.experimental.pallas import tpu as pltpu


def _tile(dim, candidates=(64, 32, 16, 8)):
    for t in candidates:
        if dim % t == 0:
            return t
    return dim


def _resize_matrix(n_in, n_out):
    o = np.arange(n_out, dtype=np.float64)
    src = (o + 0.5) * (n_in / n_out) - 0.5
    lo = np.floor(src).astype(np.int64)
    frac = (src - lo).astype(np.float32)
    lo_c = np.clip(lo, 0, n_in - 1)
    hi_c = np.clip(lo + 1, 0, n_in - 1)
    mat = np.zeros((n_out, n_in), dtype=np.float32)
    mat[np.arange(n_out), lo_c] += 1.0 - frac
    mat[np.arange(n_out), hi_c] += frac
    return mat


def _feat_kernel(wf_ref, wm_ref, x_ref, y_ref, s_ref, o_ref):
    c_in = x_ref.shape[1]
    h = x_ref.shape[2]
    w = x_ref.shape[3]
    hs = h // 3
    c_fea = wf_ref.shape[1]
    sel = s_ref[...]

    def feat(img_ref, w_off):
        x = img_ref[0].reshape(c_in, hs, 3, w)[:, :, 0, :]
        x = x.reshape(c_in * hs, w)
        xs = jnp.dot(x, sel, preferred_element_type=jnp.float32)
        acc = None
        for co in range(c_fea):
            f = wf_ref[0, co] * xs[0:hs]
            for ci in range(1, c_in):
                f = f + wf_ref[ci, co] * xs[ci * hs:(ci + 1) * hs]
            t = wm_ref[w_off + co, 0] * jnp.maximum(f, 0.0)
            acc = t if acc is None else acc + t
        return acc

    o_ref[0, 0] = feat(x_ref, 0)
    o_ref[1, 0] = feat(y_ref, c_fea)


def _features(x_img, y_img, w_fea, w_mat):
    n, c_in, h, w = x_img.shape
    hs, ws = h // 3, w // 3
    sel = np.zeros((w, ws), dtype=np.float32)
    sel[3 * np.arange(ws), np.arange(ws)] = 1.0
    return pl.pallas_call(
        _feat_kernel,
        out_shape=jax.ShapeDtypeStruct((2, n, hs, ws), jnp.float32),
        grid=(n, hs // 64),
        in_specs=[
            pl.BlockSpec(memory_space=pltpu.MemorySpace.SMEM),
            pl.BlockSpec(memory_space=pltpu.MemorySpace.SMEM),
            pl.BlockSpec((1, c_in, 192, w), lambda b, hh: (b, 0, hh, 0)),
            pl.BlockSpec((1, c_in, 192, w), lambda b, hh: (b, 0, hh, 0)),
            pl.BlockSpec((w, ws), lambda b, hh: (0, 0)),
        ],
        out_specs=pl.BlockSpec((2, 1, 64, ws), lambda b, hh: (0, b, hh, 0)),
        compiler_params=pltpu.CompilerParams(
            dimension_semantics=("parallel", "parallel")),
    )(w_fea, w_mat, x_img, y_img, jnp.asarray(sel))


def _disp_kernel(px_ref, py_ref, uh_ref, uw_ref, o_ref, *, d_low):
    px = px_ref[0, 0]
    py = py_ref[0, 0]
    uh = uh_ref[...]
    uw = uw_ref[...]
    t3, hs = uh.shape
    ws, w3 = uw.shape

    a = jnp.dot(uh, px, preferred_element_type=jnp.float32)
    b = jnp.dot(uh, py, preferred_element_type=jnp.float32)

    m_s = jnp.min(a) + jnp.min(b)
    a = a - m_s

    wpos = lax.broadcasted_iota(jnp.int32, (t3, ws), 1)
    neg_ms = -m_s
    cd = 8 if d_low % 8 == 0 else (4 if d_low % 4 == 0 else 1)
    pp = pc = None
    num = jnp.zeros((t3, w3), jnp.float32)
    den = jnp.zeros((t3, w3), jnp.float32)

    def level(d, ppv, pcv, pnv, num, den):
        p2 = pcv * pcv
        t = ppv + pcv + pnv
        num = num + p2 * ((3.0 * d + 1.0) * t + (pnv - ppv))
        den = den + p2 * t
        return num, den

    for c in range(0, d_low, cd):
        rows = []
        for j in range(cd):
            d = c + j
            r = b if d == 0 else jnp.roll(b, d, axis=1)
            rows.append(jnp.where(wpos >= d, a + r, neg_ms).astype(jnp.bfloat16))
        xc = jnp.concatenate(rows, axis=0)
        q4 = jnp.dot(xc, uw, preferred_element_type=jnp.float32)
        q4 = q4.reshape(cd, t3, w3)
        for j in range(cd):
            pn = jnp.exp2(q4[j])
            if pc is None:
                pp = pc = pn
                continue
            num, den = level(c + j - 1, pp, pc, pn, num, den)
            pp = pc
            pc = pn
    num, den = level(d_low - 1, pp, pc, pc, num, den)
    o_ref[0] = num / den


def _disp(fea2, maxdisp):
    _, n, hs, ws = fea2.shape
    d_low = maxdisp // 3
    h3, w3 = hs * 3, ws * 3
    t3 = _tile(h3, candidates=(192, 128, 64, 32, 16, 8))
    uh = jnp.asarray(_resize_matrix(hs, h3))
    uw = jnp.asarray(_resize_matrix(ws, w3).T
                     * (-1.4426950408889634 / 3.0))
    return pl.pallas_call(
        functools.partial(_disp_kernel, d_low=d_low),
        out_shape=jax.ShapeDtypeStruct((n, h3, w3), jnp.float32),
        grid=(n, h3 // t3),
        in_specs=[
            pl.BlockSpec((1, 1, hs, ws), lambda bb, hh: (0, bb, 0, 0)),
            pl.BlockSpec((1, 1, hs, ws), lambda bb, hh: (1, bb, 0, 0)),
            pl.BlockSpec((t3, hs), lambda bb, hh: (hh, 0)),
            pl.BlockSpec((ws, w3), lambda bb, hh: (0, 0)),
        ],
        out_specs=pl.BlockSpec((1, t3, w3), lambda bb, hh: (bb, hh, 0)),
        compiler_params=pltpu.CompilerParams(
            dimension_semantics=("parallel", "parallel")),
    )(fea2, fea2, uh, jnp.asarray(uw, jnp.bfloat16))


@functools.partial(jax.jit, static_argnames=("maxdisp",))
def _forward(x_img, y_img, w_fea, w_mat, *, maxdisp):
    fea2 = _features(x_img, y_img, w_fea, w_mat)
    return _disp(fea2, maxdisp)


def kernel(x_img, y_img, w_fea, w_mat):
    return _forward(x_img, y_img, w_fea, w_mat, maxdisp=192)
